# baseline (speedup 1.0000x reference)
import functools

import jax
import jax.numpy as jnp
from jax import lax
from jax.experimental import pallas as pl
from jax.experimental.pallas import tpu as pltpu
from jax.experimental.pallas import tpu_sc as plsc

EDIM_ = 32
D_ = 2 * EDIM_
B_ = 16384
NV_ = 1000000
LASTA_ = (NV_ // 128) * 128
NC_, NS_ = 2, 16
NW_ = NC_ * NS_
BPW_ = B_ // NW_
NBUF_ = 8
LOOKA_ = NBUF_ // 2


def _sc_gather_stream(ut64, it64, user_ids, item_ids):
    mesh = plsc.VectorSubcoreMesh(core_axis_name="c", subcore_axis_name="s")

    @functools.partial(
        pl.kernel,
        mesh=mesh,
        compiler_params=pltpu.CompilerParams(use_tc_tiling_on_sc=False),
        out_type=[
            jax.ShapeDtypeStruct((D_, B_), jnp.float32),
            jax.ShapeDtypeStruct((D_, B_), jnp.float32),
        ],
        scratch_types=(
            [pltpu.VMEM((BPW_ + 32,), jnp.int32)] * 2
            + [pltpu.VMEM_SHARED((D_, NS_ * BPW_), jnp.float32)]
            + [pltpu.VMEM((D_, 128), jnp.float32)] * NBUF_
            + [pltpu.SemaphoreType.DMA] * (2 * NBUF_)
        ),
    )
    def gather_kernel(ut_hbm, it_hbm, uid_hbm, iid_hbm, ue_hbm, ie_hbm,
                      uidx_v, iidx_v, sstage_v, *bufs_and_sems):
        bufs = bufs_and_sems[:NBUF_]
        sems = bufs_and_sems[NBUF_:2 * NBUF_]
        esems = bufs_and_sems[2 * NBUF_:]
        sub = lax.axis_index("s")
        wid = sub * NC_ + lax.axis_index("c")
        base = wid * BPW_
        sbase = sub * BPW_
        pltpu.sync_copy(uid_hbm.at[pl.ds(base, BPW_)],
                        uidx_v.at[pl.ds(0, BPW_)])
        pltpu.sync_copy(iid_hbm.at[pl.ds(base, BPW_)],
                        iidx_v.at[pl.ds(0, BPW_)])

        def do_table(tab_hbm, idx_v, out_hbm):
            def issue_block(idv, slot):
                rt = lax.shift_right_logical(idv, 7)

                @pl.when(idv < LASTA_)
                def _():
                    pltpu.async_copy(tab_hbm.at[:, pl.ds(rt * 128, 128)],
                                     bufs[slot], sems[slot])

                @pl.when(idv >= LASTA_)
                def _():
                    pltpu.async_copy(
                        tab_hbm.at[:, pl.ds(LASTA_, NV_ - LASTA_)],
                        bufs[slot].at[:, pl.ds(0, NV_ - LASTA_)], sems[slot])

            def wait_block(idv, slot):
                @pl.when(idv < LASTA_)
                def _():
                    pltpu.make_async_copy(
                        tab_hbm.at[:, pl.ds(0, 128)],
                        bufs[slot], sems[slot]).wait()

                @pl.when(idv >= LASTA_)
                def _():
                    pltpu.make_async_copy(
                        tab_hbm.at[:, pl.ds(0, NV_ - LASTA_)],
                        bufs[slot].at[:, pl.ds(0, NV_ - LASTA_)],
                        sems[slot]).wait()

            def issue_extract(idv, k, slot):
                lane = lax.bitwise_and(idv, 127)
                pltpu.async_copy(bufs[slot].at[:, pl.ds(lane, 1)],
                                 sstage_v.at[:, pl.ds(sbase + k, 1)],
                                 esems[slot])

            def wait_extract(slot):
                pltpu.make_async_copy(
                    bufs[slot].at[:, pl.ds(0, 1)],
                    sstage_v.at[:, pl.ds(sbase, 1)], esems[slot]).wait()

            v0 = idx_v[pl.ds(0, 16)]
            for u in range(LOOKA_):
                issue_block(v0[u], u)

            def body(t, carry):
                v = idx_v[pl.ds(16 * t, 16)]
                vb = idx_v[pl.ds(16 * t + LOOKA_, 16)]
                for u in range(16):
                    k = 16 * t + u
                    slot = u % NBUF_
                    bslot = (u + LOOKA_) % NBUF_
                    wait_block(v[u], slot)
                    issue_extract(v[u], k, slot)

                    @pl.when(jnp.logical_and(k >= LOOKA_,
                                             k + LOOKA_ < BPW_))
                    def _():
                        wait_extract(bslot)

                    @pl.when(k + LOOKA_ < BPW_)
                    def _():
                        issue_block(vb[u], bslot)

                return carry

            lax.fori_loop(0, BPW_ // 16, body, 0)
            for u in range(NBUF_):
                wait_extract(u)
            for j in range(BPW_ // 128):
                pltpu.sync_copy(sstage_v.at[:, pl.ds(sbase + 128 * j, 128)],
                                bufs[j])
                pltpu.sync_copy(bufs[j],
                                out_hbm.at[:, pl.ds(base + 128 * j, 128)])

        do_table(ut_hbm, uidx_v, ue_hbm)
        do_table(it_hbm, iidx_v, ie_hbm)

    return gather_kernel(ut64, it64, user_ids, item_ids)


def _tc_head_body(ue_ref, ie_ref, w1_ref, b1_ref, w2_ref, b2_ref,
                  w3_ref, b3_ref, o_ref):
    ue = ue_ref[...]
    ie = ie_ref[...]
    gmf = ue[:EDIM_] * ie[:EDIM_]
    x = jnp.concatenate([ue[EDIM_:], ie[EDIM_:]], axis=0)
    h1 = lax.dot_general(w1_ref[...], x, (((1,), (0,)), ((), ())),
                         preferred_element_type=jnp.float32)
    h1 = jnp.maximum(h1 + b1_ref[...], 0.0)
    h2 = lax.dot_general(w2_ref[...], h1, (((1,), (0,)), ((), ())),
                         preferred_element_type=jnp.float32)
    h2 = jnp.maximum(h2 + b2_ref[...], 0.0)
    z = jnp.concatenate([gmf, h2], axis=0)
    o = lax.dot_general(w3_ref[...], z, (((1,), (0,)), ((), ())),
                        preferred_element_type=jnp.float32)
    o_ref[...] = o + b3_ref[0]


BT_ = 2048


def _tc_head(ueT, ieT, W1, b1, W2, b2, W3, b3):
    full = lambda shape: pl.BlockSpec(shape, lambda i: (0, 0))
    out = pl.pallas_call(
        _tc_head_body,
        grid=(B_ // BT_,),
        in_specs=[
            pl.BlockSpec((D_, BT_), lambda i: (0, i)),
            pl.BlockSpec((D_, BT_), lambda i: (0, i)),
            full((EDIM_, D_)),
            full((EDIM_, 1)),
            full((EDIM_ // 2, EDIM_)),
            full((EDIM_ // 2, 1)),
            full((8, EDIM_ + EDIM_ // 2)),
            pl.BlockSpec(memory_space=pltpu.MemorySpace.SMEM),
        ],
        out_specs=pl.BlockSpec((8, BT_), lambda i: (0, i)),
        out_shape=jax.ShapeDtypeStruct((8, B_), jnp.float32),
    )(ueT, ieT, W1, b1.reshape(EDIM_, 1), W2, b2.reshape(EDIM_ // 2, 1),
      jnp.broadcast_to(W3, (8, EDIM_ + EDIM_ // 2)), b3)
    return out[0, :]


def kernel(user_ids, item_ids, user_table, item_table, W1, b1, W2, b2, W3, b3):
    uid = user_ids.astype(jnp.int32)
    iid = item_ids.astype(jnp.int32)
    ut64 = user_table.T
    it64 = item_table.T
    ueT, ieT = _sc_gather_stream(ut64, it64, uid, iid)
    return _tc_head(ueT, ieT, W1, b1, W2, b2, W3, b3)


# speedup vs baseline: 9.2259x; 9.2259x over previous
import functools

import jax
import jax.numpy as jnp
from jax import lax
from jax.experimental import pallas as pl
from jax.experimental.pallas import tpu as pltpu
from jax.experimental.pallas import tpu_sc as plsc

EDIM_ = 32
D_ = 2 * EDIM_
B_ = 16384
NC_, NS_ = 2, 16
NW_ = NC_ * NS_
BPW_ = B_ // NW_


def _sc_gather(user_table, item_table, user_ids, item_ids):
    mesh = plsc.VectorSubcoreMesh(core_axis_name="c", subcore_axis_name="s")

    @functools.partial(
        pl.kernel,
        mesh=mesh,
        compiler_params=pltpu.CompilerParams(use_tc_tiling_on_sc=False),
        out_type=[
            jax.ShapeDtypeStruct((B_, D_), jnp.float32),
            jax.ShapeDtypeStruct((B_, D_), jnp.float32),
        ],
        scratch_types=[
            pltpu.VMEM((BPW_,), jnp.int32),
            pltpu.VMEM((BPW_,), jnp.int32),
            pltpu.VMEM((BPW_, D_), jnp.float32),
            pltpu.VMEM((BPW_, D_), jnp.float32),
            pltpu.SemaphoreType.DMA,
            pltpu.SemaphoreType.DMA,
        ],
    )
    def gather_kernel(ut_hbm, it_hbm, uid_hbm, iid_hbm, ue_hbm, ie_hbm,
                      uidx_v, iidx_v, ur_v, ir_v, sem_u, sem_i):
        wid = lax.axis_index("s") * NC_ + lax.axis_index("c")
        base = wid * BPW_
        pltpu.sync_copy(uid_hbm.at[pl.ds(base, BPW_)], uidx_v)
        pltpu.sync_copy(iid_hbm.at[pl.ds(base, BPW_)], iidx_v)
        cu = pltpu.async_copy(ut_hbm.at[uidx_v], ur_v, sem_u)
        ci = pltpu.async_copy(it_hbm.at[iidx_v], ir_v, sem_i)
        cu.wait()
        pltpu.sync_copy(ur_v, ue_hbm.at[pl.ds(base, BPW_)])
        ci.wait()
        pltpu.sync_copy(ir_v, ie_hbm.at[pl.ds(base, BPW_)])

    return gather_kernel(user_table, item_table, user_ids, item_ids)


def _tc_head_body(ue_ref, ie_ref, w1_ref, b1_ref, w2_ref, b2_ref,
                  w3_ref, b3_ref, o_ref):
    ue = ue_ref[...]
    ie = ie_ref[...]
    gmf = ue[:, :EDIM_] * ie[:, :EDIM_]
    x = jnp.concatenate([ue[:, EDIM_:], ie[:, EDIM_:]], axis=1)
    h1 = lax.dot_general(x, w1_ref[...], (((1,), (1,)), ((), ())),
                         preferred_element_type=jnp.float32)
    h1 = jnp.maximum(h1 + b1_ref[...], 0.0)
    h2 = lax.dot_general(h1, w2_ref[...], (((1,), (1,)), ((), ())),
                         preferred_element_type=jnp.float32)
    h2 = jnp.maximum(h2 + b2_ref[...], 0.0)
    z = jnp.concatenate([gmf, h2], axis=1)
    o = lax.dot_general(z, w3_ref[...], (((1,), (1,)), ((), ())),
                        preferred_element_type=jnp.float32)
    o_ref[...] = o + b3_ref[0]


def _tc_head(ue, ie, W1, b1, W2, b2, W3, b3):
    out = pl.pallas_call(
        _tc_head_body,
        in_specs=[pl.BlockSpec() for _ in range(7)]
        + [pl.BlockSpec(memory_space=pltpu.MemorySpace.SMEM)],
        out_shape=jax.ShapeDtypeStruct((B_, 8), jnp.float32),
    )(ue, ie, W1, b1.reshape(1, EDIM_), W2, b2.reshape(1, EDIM_ // 2),
      jnp.broadcast_to(W3, (8, EDIM_ + EDIM_ // 2)), b3)
    return out[:, 0]


def kernel(user_ids, item_ids, user_table, item_table, W1, b1, W2, b2, W3, b3):
    uid = user_ids.astype(jnp.int32)
    iid = item_ids.astype(jnp.int32)
    ue, ie = _sc_gather(user_table, item_table, uid, iid)
    return _tc_head(ue, ie, W1, b1, W2, b2, W3, b3)
